# Initial kernel scaffold; baseline (speedup 1.0000x reference)
import jax
import jax.numpy as jnp
from jax.experimental import pallas as pl


def kernel(x, edge_index, W1, b1, u1, Wx, bx, ux):
    raise NotImplementedError("write your pallas kernel here")


# speedup vs baseline: 14.0926x; 14.0926x over previous
import functools

import jax
import jax.numpy as jnp
from jax import lax
from jax.experimental import pallas as pl
from jax.experimental.pallas import tpu as pltpu
from jax.experimental.pallas import tpu_sc as plsc

N = 10000
E = 320000
F = 128

NPAD = 10240
CH = 128
NCH = 79
NTILES = 32
EPT = NCH * CH
EPAD_TOT = NTILES * EPT
ROWS_PER_TILE = NPAD // 16
DUMMY = N

_mesh = plsc.VectorSubcoreMesh(core_axis_name="c", subcore_axis_name="s")


def _deg_kernel(dst_t):
    @functools.partial(
        pl.kernel,
        out_type=jax.ShapeDtypeStruct((2, NPAD), jnp.float32),
        mesh=_mesh,
        scratch_types=[
            pltpu.VMEM((NCH, CH), jnp.int32),
            pltpu.VMEM((CH,), jnp.float32),
            pltpu.VMEM((ROWS_PER_TILE,), jnp.float32),
            pltpu.VMEM_SHARED((NPAD,), jnp.float32),
        ],
    )
    def k(dst_hbm, deg_hbm, dst_v, ones_v, zeros_v, deg_sh):
        c = lax.axis_index("c")
        s = lax.axis_index("s")
        w = c * 16 + s
        pltpu.sync_copy(dst_hbm.at[w], dst_v)

        @pl.loop(0, CH, step=16)
        def _(i):
            ones_v.at[pl.ds(i, 16)][...] = jnp.full((16,), 1.0, jnp.float32)

        @pl.loop(0, ROWS_PER_TILE, step=16)
        def _(i):
            zeros_v.at[pl.ds(i, 16)][...] = jnp.full((16,), 0.0, jnp.float32)

        base = s * ROWS_PER_TILE
        pltpu.sync_copy(zeros_v, deg_sh.at[pl.ds(base, ROWS_PER_TILE)])
        plsc.subcore_barrier()

        @pl.loop(0, NCH)
        def _(j):
            pltpu.sync_copy(ones_v, deg_sh.at[dst_v.at[j]], add=True)

        plsc.subcore_barrier()
        pltpu.sync_copy(deg_sh.at[pl.ds(base, ROWS_PER_TILE)],
                        deg_hbm.at[c].at[pl.ds(base, ROWS_PER_TILE)])

    return k(dst_t)


def _scatter_kernel(hs_pad, src_t, dst_t):
    @functools.partial(
        pl.kernel,
        out_type=jax.ShapeDtypeStruct((2, NPAD, F), jnp.float32),
        mesh=_mesh,
        scratch_types=[
            pltpu.VMEM((NCH, CH), jnp.int32),
            pltpu.VMEM((NCH, CH), jnp.int32),
            pltpu.VMEM((CH, F), jnp.float32),
            pltpu.VMEM_SHARED((NPAD, F), jnp.float32),
        ],
    )
    def k(hs_hbm, src_hbm, dst_hbm, out_hbm, src_v, dst_v, msg_v, acc_sh):
        c = lax.axis_index("c")
        s = lax.axis_index("s")
        w = c * 16 + s
        pltpu.sync_copy(src_hbm.at[w], src_v)
        pltpu.sync_copy(dst_hbm.at[w], dst_v)

        @pl.loop(0, CH)
        def _(r):
            @pl.loop(0, F, step=16)
            def _(col):
                msg_v.at[r, pl.ds(col, 16)][...] = jnp.full((16,), 0.0,
                                                            jnp.float32)

        base = s * ROWS_PER_TILE

        @pl.loop(0, ROWS_PER_TILE, step=CH)
        def _(r):
            pltpu.sync_copy(msg_v, acc_sh.at[pl.ds(base + r, CH)])

        plsc.subcore_barrier()

        @pl.loop(0, NCH)
        def _(j):
            pltpu.sync_copy(hs_hbm.at[src_v.at[j]], msg_v)
            pltpu.sync_copy(msg_v, acc_sh.at[dst_v.at[j]], add=True)

        plsc.subcore_barrier()

        @pl.loop(0, ROWS_PER_TILE, step=CH)
        def _(r):
            pltpu.sync_copy(acc_sh.at[pl.ds(base + r, CH)],
                            out_hbm.at[c].at[pl.ds(base + r, CH)])

    return k(hs_pad, src_t, dst_t)


def _sigma_from(W, u):
    u = u / (jnp.sqrt(jnp.sum(u * u)) + 1e-12)
    v = lax.dot_general(u, W, (((1,), (0,)), ((), ())),
                        preferred_element_type=jnp.float32)
    v = v / (jnp.sqrt(jnp.sum(v * v)) + 1e-12)
    Wv = lax.dot_general(v, W, (((1,), (1,)), ((), ())),
                         preferred_element_type=jnp.float32)
    nWv = jnp.sqrt(jnp.sum(Wv * Wv))
    return (nWv * nWv) / (nWv + 1e-12)


def _prep_body(degp_ref, dinv_ref):
    deg = degp_ref[0] + degp_ref[1] + 1.0
    dinv_ref[...] = lax.rsqrt(deg)


def _prep(deg_parts):
    return pl.pallas_call(
        _prep_body,
        out_shape=jax.ShapeDtypeStruct((NPAD // F, F), jnp.float32),
    )(deg_parts.reshape(2, NPAD // F, F))


def _mm1_body(x_ref, W_ref, u_ref, dinv_ref, h_ref, hs_ref):
    sigma = _sigma_from(W_ref[...], u_ref[...])
    Wn = W_ref[...] / sigma
    h = jnp.dot(x_ref[...], Wn, preferred_element_type=jnp.float32)
    h_ref[...] = h
    hs_ref[pl.ds(0, N), :] = h * dinv_ref[...]
    hs_ref[pl.ds(N, NPAD - N), :] = jnp.zeros((NPAD - N, F), jnp.float32)


def _mm1(x, W1, u1r, dinv_col):
    return pl.pallas_call(
        _mm1_body,
        out_shape=[
            jax.ShapeDtypeStruct((N, F), jnp.float32),
            jax.ShapeDtypeStruct((NPAD, F), jnp.float32),
        ],
    )(x, W1, u1r, dinv_col)


def _combine1_body(p_ref, h1_ref, dinv_ref, b_ref, Wx_ref, ux_ref,
                   z1_ref, h2_ref, hs2_ref):
    dinv = dinv_ref[...]
    h1 = h1_ref[...]
    agg = p_ref[0, pl.ds(0, N), :] + p_ref[1, pl.ds(0, N), :]
    z1 = jnp.maximum(dinv * agg + dinv * dinv * h1 + b_ref[...], 0.0)
    z1_ref[...] = z1
    sigma = _sigma_from(Wx_ref[...], ux_ref[...])
    Wn = Wx_ref[...] / sigma
    h2 = jnp.dot(z1, Wn, preferred_element_type=jnp.float32)
    h2_ref[...] = h2
    hs2_ref[pl.ds(0, N), :] = h2 * dinv
    hs2_ref[pl.ds(N, NPAD - N), :] = jnp.zeros((NPAD - N, F), jnp.float32)


def _combine1(p1, h1, dinv_col, b1r, Wx, uxr):
    return pl.pallas_call(
        _combine1_body,
        out_shape=[
            jax.ShapeDtypeStruct((N, F), jnp.float32),
            jax.ShapeDtypeStruct((N, F), jnp.float32),
            jax.ShapeDtypeStruct((NPAD, F), jnp.float32),
        ],
    )(p1, h1, dinv_col, b1r, Wx, uxr)


def _combine2_body(p_ref, h2_ref, z1_ref, dinv_ref, b_ref, out_ref):
    dinv = dinv_ref[...]
    agg = p_ref[0, pl.ds(0, N), :] + p_ref[1, pl.ds(0, N), :]
    z2 = jnp.maximum(dinv * agg + dinv * dinv * h2_ref[...] + b_ref[...], 0.0)
    out_ref[...] = jnp.maximum(z1_ref[...], z2)


def _combine2(p2, h2, z1, dinv_col, bxr):
    return pl.pallas_call(
        _combine2_body,
        out_shape=jax.ShapeDtypeStruct((N, F), jnp.float32),
    )(p2, h2, z1, dinv_col, bxr)


def kernel(x, edge_index, W1, b1, u1, Wx, bx, ux):
    src = edge_index[0]
    dst = edge_index[1]
    pad = jnp.full((EPAD_TOT - E,), DUMMY, jnp.int32)
    src_t = jnp.concatenate([src, pad]).reshape(NTILES, NCH, CH)
    dst_t = jnp.concatenate([dst, pad]).reshape(NTILES, NCH, CH)

    u1r = u1.reshape(1, F)
    uxr = ux.reshape(1, F)
    b1r = b1.reshape(1, F)
    bxr = bx.reshape(1, F)

    deg_parts = _deg_kernel(dst_t)
    dinv_grid = _prep(deg_parts)
    dinv_col = dinv_grid.reshape(NPAD, 1)[:N]

    h1, hs1 = _mm1(x, W1, u1r, dinv_col)
    p1 = _scatter_kernel(hs1, src_t, dst_t)
    z1, h2, hs2 = _combine1(p1, h1, dinv_col, b1r, Wx, uxr)
    p2 = _scatter_kernel(hs2, src_t, dst_t)
    out = _combine2(p2, h2, z1, dinv_col, bxr)
    return out


# speedup vs baseline: 33.4463x; 2.3733x over previous
import functools

import jax
import jax.numpy as jnp
from jax import lax
from jax.experimental import pallas as pl
from jax.experimental.pallas import tpu as pltpu
from jax.experimental.pallas import tpu_sc as plsc

N = 10000
E = 320000
F = 128

NPAD = 10240
CH = 128
NCH = 80
NTILES = 32
EPT = NCH * CH
EPAD_TOT = NTILES * EPT
ROWS_PER_TILE = NPAD // 16
DUMMY = N

_mesh = plsc.VectorSubcoreMesh(core_axis_name="c", subcore_axis_name="s")


def _deg_kernel(dst_t):
    @functools.partial(
        pl.kernel,
        out_type=jax.ShapeDtypeStruct((2, NPAD), jnp.float32),
        mesh=_mesh,
        scratch_types=[
            pltpu.VMEM((NCH, CH), jnp.int32),
            pltpu.VMEM((CH,), jnp.float32),
            pltpu.VMEM((ROWS_PER_TILE,), jnp.float32),
            pltpu.VMEM_SHARED((NPAD,), jnp.float32),
        ],
    )
    def k(dst_hbm, deg_hbm, dst_v, ones_v, zeros_v, deg_sh):
        c = lax.axis_index("c")
        s = lax.axis_index("s")
        w = c * 16 + s
        pltpu.sync_copy(dst_hbm.at[w], dst_v)

        @pl.loop(0, CH, step=16)
        def _(i):
            ones_v.at[pl.ds(i, 16)][...] = jnp.full((16,), 1.0, jnp.float32)

        @pl.loop(0, ROWS_PER_TILE, step=16)
        def _(i):
            zeros_v.at[pl.ds(i, 16)][...] = jnp.full((16,), 0.0, jnp.float32)

        base = s * ROWS_PER_TILE
        pltpu.sync_copy(zeros_v, deg_sh.at[pl.ds(base, ROWS_PER_TILE)])
        plsc.subcore_barrier()

        @pl.loop(0, NCH)
        def _(j):
            pltpu.sync_copy(ones_v, deg_sh.at[dst_v.at[j]], add=True)

        plsc.subcore_barrier()
        pltpu.sync_copy(deg_sh.at[pl.ds(base, ROWS_PER_TILE)],
                        deg_hbm.at[c].at[pl.ds(base, ROWS_PER_TILE)])

    return k(dst_t)


def _scatter_kernel(hs_pad, pk_t):
    @functools.partial(
        pl.kernel,
        out_type=jax.ShapeDtypeStruct((2, NPAD, F), jnp.float32),
        mesh=_mesh,
        scratch_types=[
            pltpu.VMEM((NCH, CH), jnp.int32),
            pltpu.VMEM((2, CH), jnp.int32),
            pltpu.VMEM((2, CH), jnp.int32),
            pltpu.VMEM((CH, F), jnp.float32),
            pltpu.VMEM((CH, F), jnp.float32),
            pltpu.VMEM_SHARED((NPAD, F), jnp.float32),
            pltpu.SemaphoreType.DMA,
            pltpu.SemaphoreType.DMA,
        ],
    )
    def k(hs_hbm, pk_hbm, out_hbm, pk_v, si_v, di_v, msg0, msg1,
          acc_sh, sem0, sem1):
        c = lax.axis_index("c")
        s = lax.axis_index("s")
        w = c * 16 + s
        pltpu.sync_copy(pk_hbm.at[w], pk_v)

        def unpack_src(j, slot):
            @pl.loop(0, CH, step=16)
            def _(i):
                p = pk_v.at[j, pl.ds(i, 16)][...]
                si_v.at[slot, pl.ds(i, 16)][...] = jnp.bitwise_and(
                    p, jnp.int32(0xFFFF))

        def unpack_dst(j, slot):
            @pl.loop(0, CH, step=16)
            def _(i):
                p = pk_v.at[j, pl.ds(i, 16)][...]
                di_v.at[slot, pl.ds(i, 16)][...] = lax.shift_right_logical(
                    p, jnp.int32(16))

        @pl.loop(0, CH)
        def _(r):
            @pl.loop(0, F, step=16)
            def _(col):
                msg0.at[r, pl.ds(col, 16)][...] = jnp.full((16,), 0.0,
                                                           jnp.float32)

        base = s * ROWS_PER_TILE

        @pl.loop(0, ROWS_PER_TILE, step=CH)
        def _(r):
            pltpu.sync_copy(msg0, acc_sh.at[pl.ds(base + r, CH)])

        plsc.subcore_barrier()

        unpack_src(0, 0)
        pltpu.async_copy(hs_hbm.at[si_v.at[0]], msg0, sem0)
        unpack_src(1, 1)
        pltpu.async_copy(hs_hbm.at[si_v.at[1]], msg1, sem1)

        @pl.loop(0, NCH, step=2)
        def _(j):
            pltpu.make_async_copy(hs_hbm.at[si_v.at[0]], msg0, sem0).wait()
            unpack_dst(j, 0)
            pltpu.sync_copy(msg0, acc_sh.at[di_v.at[0]], add=True)

            @pl.when(j + 2 < NCH)
            def _():
                unpack_src(j + 2, 0)
                pltpu.async_copy(hs_hbm.at[si_v.at[0]], msg0, sem0)

            pltpu.make_async_copy(hs_hbm.at[si_v.at[1]], msg1, sem1).wait()
            unpack_dst(j + 1, 1)
            pltpu.sync_copy(msg1, acc_sh.at[di_v.at[1]], add=True)

            @pl.when(j + 3 < NCH)
            def _():
                unpack_src(j + 3, 1)
                pltpu.async_copy(hs_hbm.at[si_v.at[1]], msg1, sem1)

        plsc.subcore_barrier()

        @pl.loop(0, ROWS_PER_TILE, step=128)
        def _(r):
            pltpu.sync_copy(acc_sh.at[pl.ds(base + r, 128)],
                            out_hbm.at[c].at[pl.ds(base + r, 128)])

    return k(hs_pad, pk_t)


def _sigma_from(W, u):
    u = u / (jnp.sqrt(jnp.sum(u * u)) + 1e-12)
    v = lax.dot_general(u, W, (((1,), (0,)), ((), ())),
                        preferred_element_type=jnp.float32)
    v = v / (jnp.sqrt(jnp.sum(v * v)) + 1e-12)
    Wv = lax.dot_general(v, W, (((1,), (1,)), ((), ())),
                         preferred_element_type=jnp.float32)
    nWv = jnp.sqrt(jnp.sum(Wv * Wv))
    return (nWv * nWv) / (nWv + 1e-12)


def _prep_body(degp_ref, dinv_ref):
    deg = degp_ref[0] + degp_ref[1] + 1.0
    dinv_ref[...] = lax.rsqrt(deg)


def _prep(deg_parts):
    return pl.pallas_call(
        _prep_body,
        out_shape=jax.ShapeDtypeStruct((NPAD // F, F), jnp.float32),
    )(deg_parts.reshape(2, NPAD // F, F))


def _mm1_body(x_ref, W_ref, u_ref, dinv_ref, h_ref, hs_ref):
    sigma = _sigma_from(W_ref[...], u_ref[...])
    Wn = W_ref[...] / sigma
    h = jnp.dot(x_ref[...], Wn, preferred_element_type=jnp.float32)
    h_ref[...] = h
    hs_ref[pl.ds(0, N), :] = h * dinv_ref[...]
    hs_ref[pl.ds(N, NPAD - N), :] = jnp.zeros((NPAD - N, F), jnp.float32)


def _mm1(x, W1, u1r, dinv_col):
    return pl.pallas_call(
        _mm1_body,
        out_shape=[
            jax.ShapeDtypeStruct((N, F), jnp.float32),
            jax.ShapeDtypeStruct((NPAD, F), jnp.float32),
        ],
    )(x, W1, u1r, dinv_col)


def _combine1_body(p_ref, h1_ref, dinv_ref, b_ref, Wx_ref, ux_ref,
                   z1_ref, h2_ref, hs2_ref):
    dinv = dinv_ref[...]
    h1 = h1_ref[...]
    agg = p_ref[0, pl.ds(0, N), :] + p_ref[1, pl.ds(0, N), :]
    z1 = jnp.maximum(dinv * agg + dinv * dinv * h1 + b_ref[...], 0.0)
    z1_ref[...] = z1
    sigma = _sigma_from(Wx_ref[...], ux_ref[...])
    Wn = Wx_ref[...] / sigma
    h2 = jnp.dot(z1, Wn, preferred_element_type=jnp.float32)
    h2_ref[...] = h2
    hs2_ref[pl.ds(0, N), :] = h2 * dinv
    hs2_ref[pl.ds(N, NPAD - N), :] = jnp.zeros((NPAD - N, F), jnp.float32)


def _combine1(p1, h1, dinv_col, b1r, Wx, uxr):
    return pl.pallas_call(
        _combine1_body,
        out_shape=[
            jax.ShapeDtypeStruct((N, F), jnp.float32),
            jax.ShapeDtypeStruct((N, F), jnp.float32),
            jax.ShapeDtypeStruct((NPAD, F), jnp.float32),
        ],
    )(p1, h1, dinv_col, b1r, Wx, uxr)


def _combine2_body(p_ref, h2_ref, z1_ref, dinv_ref, b_ref, out_ref):
    dinv = dinv_ref[...]
    agg = p_ref[0, pl.ds(0, N), :] + p_ref[1, pl.ds(0, N), :]
    z2 = jnp.maximum(dinv * agg + dinv * dinv * h2_ref[...] + b_ref[...], 0.0)
    out_ref[...] = jnp.maximum(z1_ref[...], z2)


def _combine2(p2, h2, z1, dinv_col, bxr):
    return pl.pallas_call(
        _combine2_body,
        out_shape=jax.ShapeDtypeStruct((N, F), jnp.float32),
    )(p2, h2, z1, dinv_col, bxr)


def kernel(x, edge_index, W1, b1, u1, Wx, bx, ux):
    src = edge_index[0]
    dst = edge_index[1]
    pad = N + jnp.arange(EPAD_TOT - E, dtype=jnp.int32) % (NPAD - N)
    src_p = jnp.concatenate([src, pad])
    dst_p = jnp.concatenate([dst, pad])
    dst_t = dst_p.reshape(NTILES, NCH, CH)
    pk_t = ((dst_p << 16) | src_p).reshape(NTILES, NCH, CH)

    u1r = u1.reshape(1, F)
    uxr = ux.reshape(1, F)
    b1r = b1.reshape(1, F)
    bxr = bx.reshape(1, F)

    deg_parts = _deg_kernel(dst_t)
    dinv_grid = _prep(deg_parts)
    dinv_col = dinv_grid.reshape(NPAD, 1)[:N]

    h1, hs1 = _mm1(x, W1, u1r, dinv_col)
    p1 = _scatter_kernel(hs1, pk_t)
    z1, h2, hs2 = _combine1(p1, h1, dinv_col, b1r, Wx, uxr)
    p2 = _scatter_kernel(hs2, pk_t)
    out = _combine2(p2, h2, z1, dinv_col, bxr)
    return out
